# baseline (speedup 1.0000x reference)
import jax
import jax.numpy as jnp
from jax.experimental import pallas as pl
from jax.experimental.pallas import tpu as pltpu
from jax.experimental.pallas import tpu_sc as plsc


_GATHER_WINDOW = 128
_VOCAB_BLOCK = 2048


def _sc_gather_wide(table_wide, idx2d):
    batch = idx2d.shape[1]
    wide = table_wide.shape[1]
    mesh = plsc.VectorSubcoreMesh(core_axis_name="core", subcore_axis_name="subcore")

    @pl.kernel(
        out_type=jax.ShapeDtypeStruct((batch, wide), table_wide.dtype),
        mesh=mesh,
        scratch_types=[pltpu.VMEM((_GATHER_WINDOW,), jnp.int32)],
    )
    def gather_kernel(tbl_hbm, i_hbm, o_hbm, half_idx):
        def body(i_vmem, o_vmem):
            @pl.loop(0, _GATHER_WINDOW, step=16)
            def _(c):
                half_idx[pl.ds(c, 16)] = i_vmem[0, pl.ds(c, 16)] >> 1

            pltpu.sync_copy(tbl_hbm.at[half_idx], o_vmem)

        pltpu.emit_pipeline(
            body,
            grid=(batch // _GATHER_WINDOW,),
            in_specs=[
                pl.BlockSpec((1, _GATHER_WINDOW), index_map=lambda i: (0, i))
            ],
            out_specs=[
                pl.BlockSpec((_GATHER_WINDOW, wide), index_map=lambda i: (i, 0))
            ],
            core_axis_name=("core", "subcore"),
            dimension_semantics=(pltpu.PARALLEL,),
        )(i_hbm, o_hbm)

    return gather_kernel(table_wide, idx2d)


def _tc_projection(h_wide, x_col, w_out, b2d):
    batch, wide = h_wide.shape
    dim = wide // 2
    vocab = w_out.shape[0]
    grid = pl.cdiv(vocab, _VOCAB_BLOCK)

    def body(h_ref, x_ref, w_ref, b_ref, o_ref, h_scratch):
        @pl.when(pl.program_id(0) == 0)
        def _():
            odd = (x_ref[...] & 1) == 1
            h_scratch[...] = jnp.where(odd, h_ref[:, dim:], h_ref[:, :dim])

        acc = jax.lax.dot_general(
            h_scratch[...],
            w_ref[...],
            (((1,), (1,)), ((), ())),
            preferred_element_type=jnp.float32,
        )
        o_ref[...] = acc + b_ref[...]

    return pl.pallas_call(
        body,
        grid=(grid,),
        in_specs=[
            pl.BlockSpec((batch, wide), lambda i: (0, 0)),
            pl.BlockSpec((batch, 1), lambda i: (0, 0)),
            pl.BlockSpec((_VOCAB_BLOCK, dim), lambda i: (i, 0)),
            pl.BlockSpec((1, _VOCAB_BLOCK), lambda i: (0, i)),
        ],
        out_specs=pl.BlockSpec((batch, _VOCAB_BLOCK), lambda i: (0, i)),
        out_shape=jax.ShapeDtypeStruct((batch, vocab), jnp.float32),
        scratch_shapes=[pltpu.VMEM((batch, dim), jnp.float32)],
    )(h_wide, x_col, w_out, b2d)


def kernel(x, emb_table, W_out, b_out):
    vocab, dim = emb_table.shape
    table_wide = emb_table.reshape(vocab // 2, 2 * dim)
    h_wide = _sc_gather_wide(table_wide, x.reshape(1, -1))
    return _tc_projection(h_wide, x.reshape(-1, 1), W_out, b_out.reshape(1, -1))


# speedup vs baseline: 2.2253x; 2.2253x over previous
import jax
import jax.numpy as jnp
from jax.experimental import pallas as pl
from jax.experimental.pallas import tpu as pltpu
from jax.experimental.pallas import tpu_sc as plsc


_GATHER_WINDOW = 128
_VOCAB_BLOCK = 2048


def _sc_gather_wide(table_wide, idx2d):
    batch = idx2d.shape[1]
    wide = table_wide.shape[1]
    mesh = plsc.VectorSubcoreMesh(core_axis_name="core", subcore_axis_name="subcore")

    @pl.kernel(
        out_type=jax.ShapeDtypeStruct((batch, wide), table_wide.dtype),
        mesh=mesh,
        scratch_types=[pltpu.VMEM((_GATHER_WINDOW,), jnp.int32)],
    )
    def gather_kernel(tbl_hbm, i_hbm, o_hbm, half_idx):
        def body(i_vmem, o_vmem):
            @pl.loop(0, _GATHER_WINDOW, step=16)
            def _(c):
                half_idx[pl.ds(c, 16)] = i_vmem[0, pl.ds(c, 16)] >> 1

            pltpu.sync_copy(tbl_hbm.at[half_idx], o_vmem)

        pltpu.emit_pipeline(
            body,
            grid=(batch // _GATHER_WINDOW,),
            in_specs=[
                pl.BlockSpec((1, _GATHER_WINDOW), index_map=lambda i: (0, i))
            ],
            out_specs=[
                pl.BlockSpec((_GATHER_WINDOW, wide), index_map=lambda i: (i, 0))
            ],
            core_axis_name=("core", "subcore"),
            dimension_semantics=(pltpu.PARALLEL,),
        )(i_hbm, o_hbm)

    return gather_kernel(table_wide, idx2d)


def _tc_projection_t(h_wide, x_col, wt, bcol):
    batch, wide = h_wide.shape
    dim = wide // 2
    vocab = wt.shape[1]
    grid = pl.cdiv(vocab, _VOCAB_BLOCK)

    def body(h_ref, x_ref, w_ref, b_ref, o_ref, h_scratch):
        @pl.when(pl.program_id(0) == 0)
        def _():
            odd = (x_ref[...] & 1) == 1
            h_scratch[...] = jnp.where(odd, h_ref[:, dim:], h_ref[:, :dim])

        acc = jax.lax.dot_general(
            w_ref[...],
            h_scratch[...],
            (((0,), (1,)), ((), ())),
            preferred_element_type=jnp.float32,
        )
        o_ref[...] = acc + b_ref[...]

    return pl.pallas_call(
        body,
        grid=(grid,),
        in_specs=[
            pl.BlockSpec((batch, wide), lambda i: (0, 0)),
            pl.BlockSpec((batch, 1), lambda i: (0, 0)),
            pl.BlockSpec((dim, _VOCAB_BLOCK), lambda i: (0, i)),
            pl.BlockSpec((_VOCAB_BLOCK, 1), lambda i: (i, 0)),
        ],
        out_specs=pl.BlockSpec((_VOCAB_BLOCK, batch), lambda i: (i, 0)),
        out_shape=jax.ShapeDtypeStruct((vocab, batch), jnp.float32),
        scratch_shapes=[pltpu.VMEM((batch, dim), jnp.float32)],
    )(h_wide, x_col, wt, bcol)


def kernel(x, emb_table, W_out, b_out):
    vocab, dim = emb_table.shape
    table_wide = emb_table.reshape(vocab // 2, 2 * dim)
    h_wide = _sc_gather_wide(table_wide, x.reshape(1, -1))
    out_t = _tc_projection_t(
        h_wide, x.reshape(-1, 1), W_out.T, b_out.reshape(-1, 1)
    )
    return out_t.T


# speedup vs baseline: 2.8139x; 1.2645x over previous
import jax
import jax.numpy as jnp
from jax.experimental import pallas as pl
from jax.experimental.pallas import tpu as pltpu
from jax.experimental.pallas import tpu_sc as plsc


_GATHER_WINDOW = 128
_VOCAB_BLOCK = 2048


def _sc_gather_wide(table_wide, idx2d):
    batch = idx2d.shape[1]
    wide = table_wide.shape[1]
    mesh = plsc.VectorSubcoreMesh(core_axis_name="core", subcore_axis_name="subcore")

    @pl.kernel(
        out_type=jax.ShapeDtypeStruct((batch, wide), table_wide.dtype),
        mesh=mesh,
        scratch_types=[pltpu.VMEM((_GATHER_WINDOW,), jnp.int32)],
    )
    def gather_kernel(tbl_hbm, i_hbm, o_hbm, half_idx):
        def body(i_vmem, o_vmem):
            @pl.loop(0, _GATHER_WINDOW, step=16)
            def _(c):
                half_idx[pl.ds(c, 16)] = i_vmem[0, pl.ds(c, 16)] >> 1

            pltpu.sync_copy(tbl_hbm.at[half_idx], o_vmem)

        pltpu.emit_pipeline(
            body,
            grid=(batch // _GATHER_WINDOW,),
            in_specs=[
                pl.BlockSpec((1, _GATHER_WINDOW), index_map=lambda i: (0, i))
            ],
            out_specs=[
                pl.BlockSpec((_GATHER_WINDOW, wide), index_map=lambda i: (i, 0))
            ],
            core_axis_name=("core", "subcore"),
            dimension_semantics=(pltpu.PARALLEL,),
        )(i_hbm, o_hbm)

    return gather_kernel(table_wide, idx2d)


def _tc_projection_t(h_wide, x_col, wt, brow):
    batch, wide = h_wide.shape
    dim = wide // 2
    vocab = wt.shape[1]
    grid = pl.cdiv(vocab, _VOCAB_BLOCK)

    def body(h_ref, x_ref, w_ref, b_ref, o_ref, hx, wx):
        @pl.when(pl.program_id(0) == 0)
        def _():
            odd = (x_ref[...] & 1) == 1
            hx[:, 0:dim] = jnp.where(odd, h_ref[:, dim:], h_ref[:, :dim])
            hx[:, dim : dim + 1] = jnp.ones((batch, 1), jnp.float32)

        wx[0:dim, :] = w_ref[...]
        wx[dim : dim + 1, :] = b_ref[...]
        o_ref[...] = jax.lax.dot_general(
            wx[...],
            hx[...],
            (((0,), (1,)), ((), ())),
            preferred_element_type=jnp.float32,
        )

    return pl.pallas_call(
        body,
        grid=(grid,),
        in_specs=[
            pl.BlockSpec((batch, wide), lambda i: (0, 0)),
            pl.BlockSpec((batch, 1), lambda i: (0, 0)),
            pl.BlockSpec((dim, _VOCAB_BLOCK), lambda i: (0, i)),
            pl.BlockSpec((1, _VOCAB_BLOCK), lambda i: (0, i)),
        ],
        out_specs=pl.BlockSpec((_VOCAB_BLOCK, batch), lambda i: (i, 0)),
        out_shape=jax.ShapeDtypeStruct((vocab, batch), jnp.float32),
        scratch_shapes=[
            pltpu.VMEM((batch, dim + 1), jnp.float32),
            pltpu.VMEM((dim + 1, _VOCAB_BLOCK), jnp.float32),
        ],
    )(h_wide, x_col, wt, brow)


def kernel(x, emb_table, W_out, b_out):
    vocab, dim = emb_table.shape
    table_wide = emb_table.reshape(vocab // 2, 2 * dim)
    h_wide = _sc_gather_wide(table_wide, x.reshape(1, -1))
    out_t = _tc_projection_t(
        h_wide, x.reshape(-1, 1), W_out.T, b_out.reshape(1, -1)
    )
    return out_t.T


# speedup vs baseline: 3.5507x; 1.2618x over previous
import jax
import jax.numpy as jnp
from jax.experimental import pallas as pl
from jax.experimental.pallas import tpu as pltpu
from jax.experimental.pallas import tpu_sc as plsc


_VOCAB_BLOCK = 4096
_LANE = 128
_PER_TEC = 32


def _sc_gather_tiles(table_t, idx2d):
    dim, _ = table_t.shape
    batch = idx2d.shape[1]
    mesh = plsc.VectorSubcoreMesh(core_axis_name="core", subcore_axis_name="subcore")

    @pl.kernel(
        out_type=jax.ShapeDtypeStruct((batch, _LANE), table_t.dtype),
        mesh=mesh,
        compiler_params=pltpu.CompilerParams(needs_layout_passes=False),
        scratch_types=[
            pltpu.VMEM((1, _LANE), jnp.int32),
            pltpu.VMEM((dim, _LANE), jnp.float32),
            pltpu.VMEM((dim, _LANE), jnp.float32),
            pltpu.VMEM((_PER_TEC, _LANE), jnp.float32),
            pltpu.SemaphoreType.DMA,
            pltpu.SemaphoreType.DMA,
            pltpu.SemaphoreType.DMA,
            pltpu.SemaphoreType.DMA,
        ],
    )
    def gather_kernel(
        tbl_hbm, i_hbm, o_hbm, idxv, bufa, bufb, outb, semi, sema, semb, semo
    ):
        core = jax.lax.axis_index("core")
        sub = jax.lax.axis_index("subcore")
        w = sub * 2 + core
        i0 = w * _PER_TEC
        chunk = w // 4
        off = (w % 4) * _PER_TEC

        pltpu.async_copy(
            i_hbm.at[:, pl.ds(chunk * _LANE, _LANE)], idxv, semi
        ).wait()

        vecs = [idxv[0, pl.ds(off + 16 * g, 16)] for g in range(_PER_TEC // 16)]
        idxs = [v[k] for v in vecs for k in range(16)]

        def start(j, buf, sem):
            t = idxs[j] >> 7
            pltpu.async_copy(
                tbl_hbm.at[:, pl.ds(pl.multiple_of(t * _LANE, _LANE), _LANE)],
                buf,
                sem,
            )

        def finish(j, buf, sem):
            pltpu.make_async_copy(
                tbl_hbm.at[:, pl.ds(0, _LANE)], buf, sem
            ).wait()
            r = idxs[j] & (_LANE - 1)
            cols = jnp.full((16,), r, jnp.int32)
            for c in range(0, dim, 16):
                rows = jax.lax.iota(jnp.int32, 16) + c
                outb[j, pl.ds(c, 16)] = plsc.load_gather(buf, [rows, cols])

        start(0, bufa, sema)
        start(1, bufb, semb)
        for j in range(0, _PER_TEC, 2):
            finish(j, bufa, sema)
            if j + 2 < _PER_TEC:
                start(j + 2, bufa, sema)
            finish(j + 1, bufb, semb)
            if j + 3 < _PER_TEC:
                start(j + 3, bufb, semb)

        pltpu.async_copy(outb, o_hbm.at[pl.ds(i0, _PER_TEC), :], semo).wait()

    return gather_kernel(table_t, idx2d)


def _tc_projection_t(h_wide, wt, brow):
    batch, wide = h_wide.shape
    dim = wt.shape[0]
    vocab = wt.shape[1]
    grid = pl.cdiv(vocab, _VOCAB_BLOCK)

    def body(h_ref, w_ref, b_ref, o_ref, hx, wx):
        hx[:, 0:dim] = h_ref[:, 0:dim]
        hx[:, dim : dim + 1] = jnp.ones((batch, 1), jnp.float32)
        wx[0:dim, :] = w_ref[...]
        wx[dim : dim + 1, :] = b_ref[...]
        o_ref[...] = jax.lax.dot_general(
            wx[...],
            hx[...],
            (((0,), (1,)), ((), ())),
            preferred_element_type=jnp.float32,
        )

    return pl.pallas_call(
        body,
        grid=(grid,),
        in_specs=[
            pl.BlockSpec((batch, wide), lambda i: (0, 0)),
            pl.BlockSpec((dim, _VOCAB_BLOCK), lambda i: (0, i)),
            pl.BlockSpec((1, _VOCAB_BLOCK), lambda i: (0, i)),
        ],
        out_specs=pl.BlockSpec((_VOCAB_BLOCK, batch), lambda i: (i, 0)),
        out_shape=jax.ShapeDtypeStruct((vocab, batch), jnp.float32),
        compiler_params=pltpu.CompilerParams(
            dimension_semantics=("parallel",),
            vmem_limit_bytes=128 * 1024 * 1024,
        ),
        scratch_shapes=[
            pltpu.VMEM((batch, dim + 1), jnp.float32),
            pltpu.VMEM((dim + 1, _VOCAB_BLOCK), jnp.float32),
        ],
    )(h_wide, wt, brow)


def kernel(x, emb_table, W_out, b_out):
    h_wide = _sc_gather_tiles(emb_table.T, x.reshape(1, -1))
    out_t = _tc_projection_t(h_wide, W_out.T, b_out.reshape(1, -1))
    return out_t.T


# speedup vs baseline: 3.7297x; 1.0504x over previous
import jax
import jax.numpy as jnp
from jax.experimental import pallas as pl
from jax.experimental.pallas import tpu as pltpu
from jax.experimental.pallas import tpu_sc as plsc


_VOCAB_BLOCK = 4096
_LANE = 128
_PER_TEC = 32
_NBUF = 8


def _sc_gather_tiles(table_t, idx2d):
    dim, _ = table_t.shape
    batch = idx2d.shape[1]
    mesh = plsc.VectorSubcoreMesh(core_axis_name="core", subcore_axis_name="subcore")

    @pl.kernel(
        out_type=jax.ShapeDtypeStruct((batch, _LANE), table_t.dtype),
        mesh=mesh,
        compiler_params=pltpu.CompilerParams(needs_layout_passes=False),
        scratch_types=[
            pltpu.VMEM((1, _LANE), jnp.int32),
            [pltpu.VMEM((dim, _LANE), jnp.float32) for _ in range(_NBUF)],
            pltpu.VMEM((_PER_TEC, _LANE), jnp.float32),
            pltpu.SemaphoreType.DMA,
            [pltpu.SemaphoreType.DMA for _ in range(_NBUF)],
            pltpu.SemaphoreType.DMA,
        ],
    )
    def gather_kernel(
        tbl_hbm, i_hbm, o_hbm, idxv, bufs, outb, semi, sems, semo
    ):
        core = jax.lax.axis_index("core")
        sub = jax.lax.axis_index("subcore")
        w = sub * 2 + core
        i0 = w * _PER_TEC
        chunk = w // 4
        off = (w % 4) * _PER_TEC

        pltpu.async_copy(
            i_hbm.at[:, pl.ds(chunk * _LANE, _LANE)], idxv, semi
        ).wait()

        vecs = [idxv[0, pl.ds(off + 16 * g, 16)] for g in range(_PER_TEC // 16)]
        idxs = [v[k] for v in vecs for k in range(16)]

        def start(j, buf, sem):
            t = idxs[j] >> 7
            pltpu.async_copy(
                tbl_hbm.at[:, pl.ds(pl.multiple_of(t * _LANE, _LANE), _LANE)],
                buf,
                sem,
            )

        def finish(j, buf, sem):
            pltpu.make_async_copy(
                tbl_hbm.at[:, pl.ds(0, _LANE)], buf, sem
            ).wait()
            r = idxs[j] & (_LANE - 1)
            cols = jnp.full((16,), r, jnp.int32)
            for c in range(0, dim, 16):
                rows = jax.lax.iota(jnp.int32, 16) + c
                outb[j, pl.ds(c, 16)] = plsc.load_gather(buf, [rows, cols])

        for j in range(_NBUF):
            start(j, bufs[j], sems[j])
        for j in range(_PER_TEC):
            finish(j, bufs[j % _NBUF], sems[j % _NBUF])
            if j + _NBUF < _PER_TEC:
                start(j + _NBUF, bufs[j % _NBUF], sems[j % _NBUF])

        pltpu.async_copy(outb, o_hbm.at[pl.ds(i0, _PER_TEC), :], semo).wait()

    return gather_kernel(table_t, idx2d)


def _tc_projection_t(h_wide, wt, brow):
    batch, wide = h_wide.shape
    dim = wt.shape[0]
    vocab = wt.shape[1]
    grid = pl.cdiv(vocab, _VOCAB_BLOCK)

    def body(h_ref, w_ref, b_ref, o_ref, hx, wx):
        hx[:, 0:dim] = h_ref[:, 0:dim]
        hx[:, dim : dim + 1] = jnp.ones((batch, 1), jnp.float32)
        wx[0:dim, :] = w_ref[...]
        wx[dim : dim + 1, :] = b_ref[...]
        o_ref[...] = jax.lax.dot_general(
            wx[...],
            hx[...],
            (((0,), (1,)), ((), ())),
            preferred_element_type=jnp.float32,
        )

    return pl.pallas_call(
        body,
        grid=(grid,),
        in_specs=[
            pl.BlockSpec((batch, wide), lambda i: (0, 0)),
            pl.BlockSpec((dim, _VOCAB_BLOCK), lambda i: (0, i)),
            pl.BlockSpec((1, _VOCAB_BLOCK), lambda i: (0, i)),
        ],
        out_specs=pl.BlockSpec((_VOCAB_BLOCK, batch), lambda i: (i, 0)),
        out_shape=jax.ShapeDtypeStruct((vocab, batch), jnp.float32),
        compiler_params=pltpu.CompilerParams(
            dimension_semantics=("parallel",),
            vmem_limit_bytes=128 * 1024 * 1024,
        ),
        scratch_shapes=[
            pltpu.VMEM((batch, dim + 1), jnp.float32),
            pltpu.VMEM((dim + 1, _VOCAB_BLOCK), jnp.float32),
        ],
    )(h_wide, wt, brow)


def kernel(x, emb_table, W_out, b_out):
    h_wide = _sc_gather_tiles(emb_table.T, x.reshape(1, -1))
    out_t = _tc_projection_t(h_wide, W_out.T, b_out.reshape(1, -1))
    return out_t.T
